# baseline (speedup 1.0000x reference)
import jax
import jax.numpy as jnp
from jax import lax
from jax.experimental import pallas as pl
from jax.experimental.pallas import tpu as pltpu
from jax.experimental.pallas import tpu_sc as plsc

N = 10000
E = 160000
F = 128
R = 8
B = 4

NC = 2
NS = 16
L = 16
F2 = F // NC

C = 128
CPS = 80
E_PAD = NS * CPS * C
NCHUNK = E_PAD // C
N_PAD = 10240
ROWS_PER_SUB = N_PAD // NS

BN = 1000


def _xw_body(x_ref, w_ref, a_ref, out_ref):
    xb = x_ref[...]
    xv = [jnp.dot(xb, w_ref[b], preferred_element_type=jnp.float32)
          for b in range(B)]
    for r in range(R):
        acc = xv[0] * a_ref[r, 0]
        for b in range(1, B):
            acc = acc + xv[b] * a_ref[r, b]
        u = lax.bitcast_convert_type(acc, jnp.int32)
        r16 = (u + jnp.int32(0x7FFF) + ((u >> 16) & 1)) >> 16
        for c in range(NC):
            lo = r16[:, c * F2:c * F2 + 32] & jnp.int32(0xFFFF)
            hi = r16[:, c * F2 + 32:c * F2 + 64] << 16
            off = (r % 2) * F2 + c * 32
            out_ref[:, r // 2, off:off + 32] = lo | hi


def _xw_table(x, weight, w_comp):
    return pl.pallas_call(
        _xw_body,
        grid=(N // BN,),
        in_specs=[
            pl.BlockSpec((BN, F), lambda i: (i, 0)),
            pl.BlockSpec((B, F, F), lambda i: (0, 0, 0)),
            pl.BlockSpec(memory_space=pltpu.SMEM),
        ],
        out_specs=pl.BlockSpec((BN, R // 2, F), lambda i: (i, 0, 0)),
        out_shape=jax.ShapeDtypeStruct((N, R // 2, F), jnp.int32),
    )(x, weight, w_comp)


def _sc_edge_body(xw_hbm, src_hbm, et_hbm, dst_hbm, norm_hbm, out_hbm,
                  srcb, etb, dstb, normb, gidxb, rows0, rows1, rows2, rows3,
                  rowsf, acc, sem0, sem1, sem2, sem3):
    cid = lax.axis_index("c")
    sid = lax.axis_index("s")

    def _zrow(i, carry):
        for j in range(F2 // L):
            rowsf[i, pl.ds(j * L, L)] = jnp.zeros((L,), jnp.float32)
        return carry
    lax.fori_loop(0, C, _zrow, 0)
    for t in range(ROWS_PER_SUB // C):
        pltpu.sync_copy(rowsf, acc.at[pl.ds(sid * ROWS_PER_SUB + t * C, C)])

    mbase = sid * CPS
    pltpu.sync_copy(src_hbm.at[pl.ds(mbase, CPS)], srcb)
    pltpu.sync_copy(et_hbm.at[pl.ds(mbase, CPS)], etb)
    pltpu.sync_copy(dst_hbm.at[pl.ds(mbase, CPS)], dstb)
    pltpu.sync_copy(norm_hbm.at[pl.ds(mbase, CPS)], normb)
    def _gidx(q, carry):
        for g in range(C // L):
            sl = pl.ds(g * L, L)
            gidxb[q, sl] = (srcb[q, sl] * R + etb[q, sl]) * 2 + cid
        return carry
    lax.fori_loop(0, CPS, _gidx, 0)
    plsc.subcore_barrier()

    bufs = (rows0, rows1, rows2, rows3)
    sems = (sem0, sem1, sem2, sem3)

    def _issue(q, k):
        pltpu.async_copy(xw_hbm.at[gidxb.at[q]], bufs[k], sems[k])

    def _drain(q, k):
        pltpu.make_async_copy(xw_hbm.at[gidxb.at[q]], bufs[k], sems[k]).wait()

    def _process(q, k):
        rowsb = bufs[k]

        def _scale(g, c2):
            nv16 = normb[q, pl.ds(g * L, L)]
            for kk in range(L):
                nv = nv16[kk]
                r = g * L + kk
                for gp in range(2):
                    v = rowsb[r, pl.ds(gp * L, L)]
                    lo = plsc.bitcast(lax.shift_left(v, 16), jnp.float32) * nv
                    hi = plsc.bitcast(v & jnp.int32(-65536), jnp.float32) * nv
                    rowsf[r, pl.ds(gp * L, L)] = lo
                    rowsf[r, pl.ds(2 * L + gp * L, L)] = hi
            return c2
        lax.fori_loop(0, C // L, _scale, 0)
        pltpu.sync_copy(rowsf, acc.at[dstb.at[q]], add=True)

    NBUF = 4
    for b in range(NBUF - 1):
        _issue(b, b)

    def _quad(g, carry):
        for b in range(NBUF):
            q = g * NBUF + b

            @pl.when(q + NBUF - 1 < CPS)
            def _():
                _issue(q + NBUF - 1, (b + NBUF - 1) % NBUF)
            _drain(q, b)
            _process(q, b)
        return carry

    lax.fori_loop(0, CPS // NBUF, _quad, 0)
    plsc.subcore_barrier()

    for t in range(ROWS_PER_SUB // C):
        sl = pl.ds(sid * ROWS_PER_SUB + t * C, C)
        pltpu.sync_copy(acc.at[sl], rowsf)
        pltpu.sync_copy(rowsf, out_hbm.at[sl, pl.ds(cid * F2, F2)])


def _sc_edge(xw_flat, src2, et2, dst2, norm2):
    mesh = plsc.VectorSubcoreMesh(core_axis_name="c", subcore_axis_name="s",
                                  num_cores=NC, num_subcores=NS)
    fn = pl.kernel(
        _sc_edge_body,
        out_type=jax.ShapeDtypeStruct((N_PAD, F), jnp.float32),
        mesh=mesh,
        compiler_params=pltpu.CompilerParams(use_tc_tiling_on_sc=False,
                                             needs_layout_passes=False),
        scratch_types=[
            pltpu.VMEM((CPS, C), jnp.int32),
            pltpu.VMEM((CPS, C), jnp.int32),
            pltpu.VMEM((CPS, C), jnp.int32),
            pltpu.VMEM((CPS, C), jnp.float32),
            pltpu.VMEM((CPS, C), jnp.int32),
            pltpu.VMEM((C, F2 // 2), jnp.int32),
            pltpu.VMEM((C, F2 // 2), jnp.int32),
            pltpu.VMEM((C, F2 // 2), jnp.int32),
            pltpu.VMEM((C, F2 // 2), jnp.int32),
            pltpu.VMEM((C, F2), jnp.float32),
            pltpu.VMEM_SHARED((N_PAD, F2), jnp.float32),
            pltpu.SemaphoreType.DMA,
            pltpu.SemaphoreType.DMA,
            pltpu.SemaphoreType.DMA,
            pltpu.SemaphoreType.DMA,
        ],
    )
    return fn(xw_flat, src2, et2, dst2, norm2)


def _comb_body(x_ref, lw_ref, b_ref, p_ref, out_ref):
    d = jnp.dot(x_ref[...], lw_ref[...], preferred_element_type=jnp.float32)
    out_ref[...] = d + b_ref[...] + p_ref[...]


def _combine(x, loop_weight, h_bias, partials):
    return pl.pallas_call(
        _comb_body,
        grid=(N // BN,),
        in_specs=[
            pl.BlockSpec((BN, F), lambda i: (i, 0)),
            pl.BlockSpec((F, F), lambda i: (0, 0)),
            pl.BlockSpec((1, F), lambda i: (0, 0)),
            pl.BlockSpec((BN, F), lambda i: (i, 0)),
        ],
        out_specs=pl.BlockSpec((BN, F), lambda i: (i, 0)),
        out_shape=jax.ShapeDtypeStruct((N, F), jnp.float32),
    )(x, loop_weight, h_bias.reshape(1, F), partials)


def kernel(x, edge_index, etype, norm, weight, w_comp, loop_weight, h_bias):
    src = edge_index[0].astype(jnp.int32)
    dst = edge_index[1].astype(jnp.int32)
    et = etype.astype(jnp.int32)
    normf = norm.reshape(E)
    pad = E_PAD - E
    src2 = jnp.pad(src, (0, pad)).reshape(NCHUNK, C)
    dst2 = jnp.pad(dst, (0, pad)).reshape(NCHUNK, C)
    et2 = jnp.pad(et, (0, pad)).reshape(NCHUNK, C)
    norm2 = jnp.pad(normf, (0, pad)).reshape(NCHUNK, C)
    xw = _xw_table(x, weight, w_comp)
    xw_flat = xw.reshape(NC * N * R, F2 // 2)
    partials = _sc_edge(xw_flat, src2, et2, dst2, norm2)
    return _combine(x, loop_weight, h_bias, partials)


# speedup vs baseline: 1.2093x; 1.2093x over previous
import jax
import jax.numpy as jnp
from jax import lax
from jax.experimental import pallas as pl
from jax.experimental.pallas import tpu as pltpu
from jax.experimental.pallas import tpu_sc as plsc

N = 10000
E = 160000
F = 128
R = 8
B = 4

NC = 2
NS = 16
L = 16
F2 = F // NC

C = 128
CPS = 80
E_PAD = NS * CPS * C
NCHUNK = E_PAD // C
N_PAD = 10240
ROWS_PER_SUB = N_PAD // NS

BN = 400


def _xw_body(x_ref, w_ref, a_ref, out_ref):
    xb = x_ref[...]
    xv = [jnp.dot(xb, w_ref[b], preferred_element_type=jnp.float32)
          for b in range(B)]
    for r in range(R):
        acc = xv[0] * a_ref[r, 0]
        for b in range(1, B):
            acc = acc + xv[b] * a_ref[r, b]
        out_ref[:, r, :] = acc


def _xw_table(x, weight, w_comp):
    return pl.pallas_call(
        _xw_body,
        grid=(N // BN,),
        in_specs=[
            pl.BlockSpec((BN, F), lambda i: (i, 0)),
            pl.BlockSpec((B, F, F), lambda i: (0, 0, 0)),
            pl.BlockSpec(memory_space=pltpu.SMEM),
        ],
        out_specs=pl.BlockSpec((BN, R, F), lambda i: (i, 0, 0)),
        out_shape=jax.ShapeDtypeStruct((N, R, F), jnp.float32),
    )(x, weight, w_comp)


def _sc_edge_body(xw_hbm, src_hbm, et_hbm, dst_hbm, norm_hbm, out_hbm,
                  srcb, etb, dstb, normb, gidxb, rows0, rows1, rows2, rows3,
                  acc, sem0, sem1, sem2, sem3):
    cid = lax.axis_index("c")
    sid = lax.axis_index("s")

    def _zrow(i, carry):
        for j in range(F2 // L):
            rows0[i, pl.ds(j * L, L)] = jnp.zeros((L,), jnp.float32)
        return carry
    lax.fori_loop(0, C, _zrow, 0)
    for t in range(ROWS_PER_SUB // C):
        pltpu.sync_copy(rows0, acc.at[pl.ds(sid * ROWS_PER_SUB + t * C, C)])

    mbase = sid * CPS
    pltpu.sync_copy(src_hbm.at[pl.ds(mbase, CPS)], srcb)
    pltpu.sync_copy(et_hbm.at[pl.ds(mbase, CPS)], etb)
    pltpu.sync_copy(dst_hbm.at[pl.ds(mbase, CPS)], dstb)
    pltpu.sync_copy(norm_hbm.at[pl.ds(mbase, CPS)], normb)
    def _gidx(q, carry):
        for g in range(C // L):
            sl = pl.ds(g * L, L)
            gidxb[q, sl] = (srcb[q, sl] * R + etb[q, sl]) * 2 + cid
        return carry
    lax.fori_loop(0, CPS, _gidx, 0)
    plsc.subcore_barrier()

    bufs = (rows0, rows1, rows2, rows3)
    sems = (sem0, sem1, sem2, sem3)

    def _issue(q, k):
        pltpu.async_copy(xw_hbm.at[gidxb.at[q]], bufs[k], sems[k])

    def _drain(q, k):
        pltpu.make_async_copy(xw_hbm.at[gidxb.at[q]], bufs[k], sems[k]).wait()

    def _process(q, k):
        rows = bufs[k]

        def _scale(g, c2):
            nv16 = normb[q, pl.ds(g * L, L)]
            for kk in range(L):
                nv = nv16[kk]
                r = g * L + kk
                for j in range(F2 // L):
                    sl = pl.ds(j * L, L)
                    rows[r, sl] = rows[r, sl] * nv
            return c2
        lax.fori_loop(0, C // L, _scale, 0)
        pltpu.sync_copy(rows, acc.at[dstb.at[q]], add=True)

    NBUF = 4
    for b in range(NBUF - 1):
        _issue(b, b)

    def _quad(g, carry):
        for b in range(NBUF):
            q = g * NBUF + b

            @pl.when(q + NBUF - 1 < CPS)
            def _():
                _issue(q + NBUF - 1, (b + NBUF - 1) % NBUF)
            _drain(q, b)
            _process(q, b)
        return carry

    lax.fori_loop(0, CPS // NBUF, _quad, 0)
    plsc.subcore_barrier()

    for t in range(ROWS_PER_SUB // C):
        sl = pl.ds(sid * ROWS_PER_SUB + t * C, C)
        pltpu.sync_copy(acc.at[sl], rows0)
        pltpu.sync_copy(rows0, out_hbm.at[sl, pl.ds(cid * F2, F2)])


def _sc_edge(xw_flat, src2, et2, dst2, norm2):
    mesh = plsc.VectorSubcoreMesh(core_axis_name="c", subcore_axis_name="s",
                                  num_cores=NC, num_subcores=NS)
    fn = pl.kernel(
        _sc_edge_body,
        out_type=jax.ShapeDtypeStruct((N_PAD, F), jnp.float32),
        mesh=mesh,
        compiler_params=pltpu.CompilerParams(use_tc_tiling_on_sc=False),
        scratch_types=[
            pltpu.VMEM((CPS, C), jnp.int32),
            pltpu.VMEM((CPS, C), jnp.int32),
            pltpu.VMEM((CPS, C), jnp.int32),
            pltpu.VMEM((CPS, C), jnp.float32),
            pltpu.VMEM((CPS, C), jnp.int32),
            pltpu.VMEM((C, F2), jnp.float32),
            pltpu.VMEM((C, F2), jnp.float32),
            pltpu.VMEM((C, F2), jnp.float32),
            pltpu.VMEM((C, F2), jnp.float32),
            pltpu.VMEM_SHARED((N_PAD, F2), jnp.float32),
            pltpu.SemaphoreType.DMA,
            pltpu.SemaphoreType.DMA,
            pltpu.SemaphoreType.DMA,
            pltpu.SemaphoreType.DMA,
        ],
    )
    return fn(xw_flat, src2, et2, dst2, norm2)


def _comb_body(x_ref, lw_ref, b_ref, p_ref, out_ref):
    d = jnp.dot(x_ref[...], lw_ref[...], preferred_element_type=jnp.float32)
    out_ref[...] = d + b_ref[...] + p_ref[...]


def _combine(x, loop_weight, h_bias, partials):
    return pl.pallas_call(
        _comb_body,
        grid=(N // BN,),
        in_specs=[
            pl.BlockSpec((BN, F), lambda i: (i, 0)),
            pl.BlockSpec((F, F), lambda i: (0, 0)),
            pl.BlockSpec((1, F), lambda i: (0, 0)),
            pl.BlockSpec((BN, F), lambda i: (i, 0)),
        ],
        out_specs=pl.BlockSpec((BN, F), lambda i: (i, 0)),
        out_shape=jax.ShapeDtypeStruct((N, F), jnp.float32),
    )(x, loop_weight, h_bias.reshape(1, F), partials)


def kernel(x, edge_index, etype, norm, weight, w_comp, loop_weight, h_bias):
    src = edge_index[0].astype(jnp.int32)
    dst = edge_index[1].astype(jnp.int32)
    et = etype.astype(jnp.int32)
    normf = norm.reshape(E)
    pad = E_PAD - E
    src2 = jnp.pad(src, (0, pad)).reshape(NCHUNK, C)
    dst2 = jnp.pad(dst, (0, pad)).reshape(NCHUNK, C)
    et2 = jnp.pad(et, (0, pad)).reshape(NCHUNK, C)
    norm2 = jnp.pad(normf, (0, pad)).reshape(NCHUNK, C)
    xw = _xw_table(x, weight, w_comp)
    xw_flat = xw.reshape(NC * N * R, F2)
    partials = _sc_edge(xw_flat, src2, et2, dst2, norm2)
    return _combine(x, loop_weight, h_bias, partials)


# speedup vs baseline: 1.2491x; 1.0328x over previous
import jax
import jax.numpy as jnp
from jax import lax
from jax.experimental import pallas as pl
from jax.experimental.pallas import tpu as pltpu
from jax.experimental.pallas import tpu_sc as plsc

N = 10000
E = 160000
F = 128
R = 8
B = 4

NC = 2
NS = 16
L = 16
F2 = F // NC

C = 128
CPS = 80
E_PAD = NS * CPS * C
NCHUNK = E_PAD // C
N_PAD = 10240
ROWS_PER_SUB = N_PAD // NS

BN = 1000


def _xw_body(x_ref, w_ref, a_ref, out_ref):
    xb = x_ref[...]
    xv = [jnp.dot(xb, w_ref[b], preferred_element_type=jnp.float32)
          for b in range(B)]
    for r in range(R):
        acc = xv[0] * a_ref[r, 0]
        for b in range(1, B):
            acc = acc + xv[b] * a_ref[r, b]
        out_ref[:, r, :] = acc


def _xw_table(x, weight, w_comp):
    return pl.pallas_call(
        _xw_body,
        grid=(N // BN,),
        in_specs=[
            pl.BlockSpec((BN, F), lambda i: (i, 0)),
            pl.BlockSpec((B, F, F), lambda i: (0, 0, 0)),
            pl.BlockSpec(memory_space=pltpu.SMEM),
        ],
        out_specs=pl.BlockSpec((BN, R, F), lambda i: (i, 0, 0)),
        out_shape=jax.ShapeDtypeStruct((N, R, F), jnp.float32),
    )(x, weight, w_comp)


def _sc_edge_body(xw_hbm, src_hbm, et_hbm, dst_hbm, norm_hbm, out_hbm,
                  srcb, etb, dstb, normb, gidxb, rows0, rows1, rows2, rows3,
                  acc, sem0, sem1, sem2, sem3):
    cid = lax.axis_index("c")
    sid = lax.axis_index("s")

    def _zrow(i, carry):
        for j in range(F2 // L):
            rows0[i, pl.ds(j * L, L)] = jnp.zeros((L,), jnp.float32)
        return carry
    lax.fori_loop(0, C, _zrow, 0)
    for t in range(ROWS_PER_SUB // C):
        pltpu.sync_copy(rows0, acc.at[pl.ds(sid * ROWS_PER_SUB + t * C, C)])

    mbase = sid * CPS
    pltpu.sync_copy(src_hbm.at[pl.ds(mbase, CPS)], srcb)
    pltpu.sync_copy(et_hbm.at[pl.ds(mbase, CPS)], etb)
    pltpu.sync_copy(dst_hbm.at[pl.ds(mbase, CPS)], dstb)
    pltpu.sync_copy(norm_hbm.at[pl.ds(mbase, CPS)], normb)
    def _gidx(q, carry):
        for g in range(C // L):
            sl = pl.ds(g * L, L)
            gidxb[q, sl] = (srcb[q, sl] * R + etb[q, sl]) * 2 + cid
        return carry
    lax.fori_loop(0, CPS, _gidx, 0)
    plsc.subcore_barrier()

    bufs = (rows0, rows1, rows2, rows3)
    sems = (sem0, sem1, sem2, sem3)

    def _issue(q, k):
        pltpu.async_copy(xw_hbm.at[gidxb.at[q]], bufs[k], sems[k])

    def _drain(q, k):
        pltpu.make_async_copy(xw_hbm.at[gidxb.at[q]], bufs[k], sems[k]).wait()

    def _process(q, k):
        rows = bufs[k]

        def _scale(g, c2):
            nv16 = normb[q, pl.ds(g * L, L)]
            for kk in range(L):
                nv = nv16[kk]
                r = g * L + kk
                for j in range(F2 // L):
                    sl = pl.ds(j * L, L)
                    rows[r, sl] = rows[r, sl] * nv
            return c2
        lax.fori_loop(0, C // L, _scale, 0)
        pltpu.sync_copy(rows, acc.at[dstb.at[q]], add=True)

    NBUF = 4
    for b in range(NBUF - 1):
        _issue(b, b)

    def _quad(g, carry):
        for b in range(NBUF):
            q = g * NBUF + b

            @pl.when(q + NBUF - 1 < CPS)
            def _():
                _issue(q + NBUF - 1, (b + NBUF - 1) % NBUF)
            _drain(q, b)
            _process(q, b)
        return carry

    lax.fori_loop(0, CPS // NBUF, _quad, 0)
    plsc.subcore_barrier()

    for t in range(ROWS_PER_SUB // C):
        sl = pl.ds(sid * ROWS_PER_SUB + t * C, C)
        pltpu.sync_copy(acc.at[sl], rows0)
        pltpu.sync_copy(rows0, out_hbm.at[sl, pl.ds(cid * F2, F2)])


def _sc_edge(xw_flat, src2, et2, dst2, norm2):
    mesh = plsc.VectorSubcoreMesh(core_axis_name="c", subcore_axis_name="s",
                                  num_cores=NC, num_subcores=NS)
    fn = pl.kernel(
        _sc_edge_body,
        out_type=jax.ShapeDtypeStruct((N_PAD, F), jnp.float32),
        mesh=mesh,
        compiler_params=pltpu.CompilerParams(use_tc_tiling_on_sc=False),
        scratch_types=[
            pltpu.VMEM((CPS, C), jnp.int32),
            pltpu.VMEM((CPS, C), jnp.int32),
            pltpu.VMEM((CPS, C), jnp.int32),
            pltpu.VMEM((CPS, C), jnp.float32),
            pltpu.VMEM((CPS, C), jnp.int32),
            pltpu.VMEM((C, F2), jnp.float32),
            pltpu.VMEM((C, F2), jnp.float32),
            pltpu.VMEM((C, F2), jnp.float32),
            pltpu.VMEM((C, F2), jnp.float32),
            pltpu.VMEM_SHARED((N_PAD, F2), jnp.float32),
            pltpu.SemaphoreType.DMA,
            pltpu.SemaphoreType.DMA,
            pltpu.SemaphoreType.DMA,
            pltpu.SemaphoreType.DMA,
        ],
    )
    return fn(xw_flat, src2, et2, dst2, norm2)


def _self_body(x_ref, lw_ref, b_ref, out_ref):
    d = jnp.dot(x_ref[...], lw_ref[...], preferred_element_type=jnp.float32)
    out_ref[...] = d + b_ref[...]


def _selfloop(x, loop_weight, h_bias):
    return pl.pallas_call(
        _self_body,
        grid=(N // BN,),
        in_specs=[
            pl.BlockSpec((BN, F), lambda i: (i, 0)),
            pl.BlockSpec((F, F), lambda i: (0, 0)),
            pl.BlockSpec((1, F), lambda i: (0, 0)),
        ],
        out_specs=pl.BlockSpec((BN, F), lambda i: (i, 0)),
        out_shape=jax.ShapeDtypeStruct((N, F), jnp.float32),
    )(x, loop_weight, h_bias.reshape(1, F))


def _add_body(s_ref, p_ref, out_ref):
    out_ref[...] = s_ref[...] + p_ref[...]


def _combine(selfloop, partials):
    return pl.pallas_call(
        _add_body,
        grid=(N // BN,),
        in_specs=[
            pl.BlockSpec((BN, F), lambda i: (i, 0)),
            pl.BlockSpec((BN, F), lambda i: (i, 0)),
        ],
        out_specs=pl.BlockSpec((BN, F), lambda i: (i, 0)),
        out_shape=jax.ShapeDtypeStruct((N, F), jnp.float32),
    )(selfloop, partials)


def kernel(x, edge_index, etype, norm, weight, w_comp, loop_weight, h_bias):
    src = edge_index[0].astype(jnp.int32)
    dst = edge_index[1].astype(jnp.int32)
    et = etype.astype(jnp.int32)
    normf = norm.reshape(E)
    pad = E_PAD - E
    src2 = jnp.pad(src, (0, pad)).reshape(NCHUNK, C)
    dst2 = jnp.pad(dst, (0, pad)).reshape(NCHUNK, C)
    et2 = jnp.pad(et, (0, pad)).reshape(NCHUNK, C)
    norm2 = jnp.pad(normf, (0, pad)).reshape(NCHUNK, C)
    xw = _xw_table(x, weight, w_comp)
    xw_flat = xw.reshape(NC * N * R, F2)
    partials = _sc_edge(xw_flat, src2, et2, dst2, norm2)
    sl = _selfloop(x, loop_weight, h_bias)
    return _combine(sl, partials)
